# Initial kernel scaffold; baseline (speedup 1.0000x reference)
import jax
import jax.numpy as jnp
from jax.experimental import pallas as pl


def kernel(x, edge_index, distances, edge_attr, params):
    raise NotImplementedError("write your pallas kernel here")


# speedup vs baseline: 3.3020x; 3.3020x over previous
import functools

import jax
import jax.numpy as jnp
import numpy as np
from jax import lax
from jax.experimental import pallas as pl
from jax.experimental.pallas import tpu as pltpu
from jax.experimental.pallas import tpu_sc as plsc

_N = 10000
_E = 320000
_HID = 128
_NF = 64
_CUTOFF = 5.0
_EPS = 1e-3

_NC = 2
_NS = 16
_NW = _NC * _NS
_CH = 128
_CHUNKS = 80
_SEG = 16
_EPT = _CH * _CHUNKS
_EP = _EPT * _NW
_NPAD = 10240
_ROWS_PER_TILE = _NPAD // _NS
_ZROWS = _CH

_RBLK = 1000


def _softplus(x):
    return jnp.maximum(x, 0.0) + jnp.log1p(jnp.exp(-jnp.abs(x)))


def _embed_body(x_ref, w_ref, b_ref, o_ref):
    o_ref[...] = jnp.dot(x_ref[...], w_ref[...],
                         preferred_element_type=jnp.float32) + b_ref[...]


def _embed(x, w, b):
    grid = _N // _RBLK
    return pl.pallas_call(
        _embed_body,
        grid=(grid,),
        in_specs=[
            pl.BlockSpec((_RBLK, _HID), lambda i: (i, 0)),
            pl.BlockSpec((_HID, _HID), lambda i: (0, 0)),
            pl.BlockSpec((1, _HID), lambda i: (0, 0)),
        ],
        out_specs=pl.BlockSpec((_RBLK, _HID), lambda i: (i, 0)),
        out_shape=jax.ShapeDtypeStruct((_N, _HID), jnp.float32),
    )(x, w, b)


def _wcalc_body(d_ref, a_ref, b_ref, c_ref, b2_ref, o_ref):
    d = d_ref[...]
    scaled = d * (2.0 / _CUTOFF) - 1.0
    cut = 0.5 * (jnp.cos(d * (np.pi / _CUTOFF)) + 1.0)
    cut = jnp.where(d <= _CUTOFF, cut, 0.0)
    for p in range(3):
        acc = jnp.zeros_like(d)
        for j in range(_NF):
            acc = acc + jnp.tanh(scaled * a_ref[p, j] + b_ref[p, j]) * c_ref[p, j]
        o_ref[p, :, :] = (acc + b2_ref[p, 0]) * cut


def _wcalc(d2d, a, b, c, b2):
    rows = _EP // _HID
    blk = 128
    grid = rows // blk
    return pl.pallas_call(
        _wcalc_body,
        grid=(grid,),
        in_specs=[
            pl.BlockSpec((blk, _HID), lambda i: (i, 0)),
            pl.BlockSpec((3, _NF), lambda i: (0, 0)),
            pl.BlockSpec((3, _NF), lambda i: (0, 0)),
            pl.BlockSpec((3, _NF), lambda i: (0, 0)),
            pl.BlockSpec((3, 128), lambda i: (0, 0)),
        ],
        out_specs=pl.BlockSpec((3, blk, _HID), lambda i: (0, i, 0)),
        out_shape=jax.ShapeDtypeStruct((3, rows, _HID), jnp.float32),
    )(d2d, a, b, c, b2)


def _sc_body(h_hbm, col_hbm, row_hbm, w_hbm, out_hbm,
             col_v, row_v, w_v, buf_a, buf_b, gsem_a, gsem_b, agg_sh):
    c = lax.axis_index("c")
    s = lax.axis_index("s")
    wid = c * _NS + s

    zero16 = jnp.zeros((16,), jnp.float32)

    def _zrow(i, _):
        for g in range(8):
            buf_b[i, pl.ds(g * 16, 16)] = zero16
        return 0

    lax.fori_loop(0, _CH, _zrow, 0)
    base = s * _ROWS_PER_TILE
    for k in range(_ROWS_PER_TILE // _ZROWS):
        pltpu.sync_copy(buf_b, agg_sh.at[pl.ds(base + k * _ZROWS, _ZROWS)])
    plsc.subcore_barrier()

    def _scale(buf, j):
        def _grp(g, _):
            w16 = w_v[j, pl.ds(g * 16, 16)]
            for e16 in range(16):
                sp = jnp.full((16,), w16[e16])
                e = g * 16 + e16
                for f in range(8):
                    sl = pl.ds(f * 16, 16)
                    buf[e, sl] = buf[e, sl] * sp
            return 0
        lax.fori_loop(0, 8, _grp, 0)

    def _segment(seg, _):
        lo = seg * _SEG
        pltpu.sync_copy(col_hbm.at[wid, pl.ds(lo, _SEG)], col_v)
        pltpu.sync_copy(row_hbm.at[wid, pl.ds(lo, _SEG)], row_v)
        pltpu.sync_copy(w_hbm.at[wid, pl.ds(lo, _SEG)], w_v)
        pltpu.async_copy(h_hbm.at[col_v.at[0]], buf_a, gsem_a)

        def _pair(i, _):
            j0 = 2 * i
            gather_b = pltpu.async_copy(h_hbm.at[col_v.at[j0 + 1]], buf_b, gsem_b)
            pltpu.make_async_copy(h_hbm.at[col_v.at[j0]], buf_a, gsem_a).wait()
            _scale(buf_a, j0)
            pltpu.sync_copy(buf_a, agg_sh.at[row_v.at[j0]], add=True)

            @pl.when(i < _SEG // 2 - 1)
            def _():
                pltpu.async_copy(h_hbm.at[col_v.at[j0 + 2]], buf_a, gsem_a)

            gather_b.wait()
            _scale(buf_b, j0 + 1)
            pltpu.sync_copy(buf_b, agg_sh.at[row_v.at[j0 + 1]], add=True)
            return 0

        lax.fori_loop(0, _SEG // 2, _pair, 0)
        return 0

    lax.fori_loop(0, _CHUNKS // _SEG, _segment, 0)
    plsc.subcore_barrier()

    for k in range(_ROWS_PER_TILE // _ZROWS):
        lo = base + k * _ZROWS
        pltpu.sync_copy(agg_sh.at[pl.ds(lo, _ZROWS)], buf_a)
        pltpu.sync_copy(buf_a, out_hbm.at[c, pl.ds(lo, _ZROWS)])


def _sc_scatter(h, col3, row3, w3):
    mesh = plsc.VectorSubcoreMesh(core_axis_name="c", subcore_axis_name="s")
    return pl.kernel(
        _sc_body,
        out_type=jax.ShapeDtypeStruct((_NC, _NPAD, _HID), jnp.float32),
        mesh=mesh,
        scratch_types=[
            pltpu.VMEM((_SEG, _CH), jnp.int32),
            pltpu.VMEM((_SEG, _CH), jnp.int32),
            pltpu.VMEM((_SEG, _CH), jnp.float32),
            pltpu.VMEM((_CH, _HID), jnp.float32),
            pltpu.VMEM((_CH, _HID), jnp.float32),
            pltpu.SemaphoreType.DMA,
            pltpu.SemaphoreType.DMA,
            pltpu.VMEM_SHARED((_NPAD, _HID), jnp.float32),
        ],
    )(h, col3, row3, w3)


def _mlp_body(p_ref, h_ref, w1_ref, w2_ref, b1_ref, sc_ref, sh_ref,
              o_ref, sum_ref):
    a = p_ref[0, :, :] + p_ref[1, :, :]
    t = jnp.dot(a, w1_ref[...], preferred_element_type=jnp.float32) + b1_ref[...]
    u = _softplus(t)
    v = jnp.dot(u, w2_ref[...], preferred_element_type=jnp.float32)
    hn = h_ref[...] + v * sc_ref[...] + sh_ref[...]
    o_ref[...] = hn

    @pl.when(pl.program_id(0) == 0)
    def _():
        sum_ref[...] = jnp.zeros_like(sum_ref)

    sum_ref[...] += jnp.sum(hn, axis=0, keepdims=True)


def _mlp(parts, h, w1, w2, b1, scale, shift2):
    grid = _N // _RBLK
    return pl.pallas_call(
        _mlp_body,
        grid=(grid,),
        in_specs=[
            pl.BlockSpec((_NC, _RBLK, _HID), lambda i: (0, i, 0)),
            pl.BlockSpec((_RBLK, _HID), lambda i: (i, 0)),
            pl.BlockSpec((_HID, _HID), lambda i: (0, 0)),
            pl.BlockSpec((_HID, _HID), lambda i: (0, 0)),
            pl.BlockSpec((1, _HID), lambda i: (0, 0)),
            pl.BlockSpec((1, _HID), lambda i: (0, 0)),
            pl.BlockSpec((1, _HID), lambda i: (0, 0)),
        ],
        out_specs=[
            pl.BlockSpec((_RBLK, _HID), lambda i: (i, 0)),
            pl.BlockSpec((1, _HID), lambda i: (0, 0)),
        ],
        out_shape=[
            jax.ShapeDtypeStruct((_N, _HID), jnp.float32),
            jax.ShapeDtypeStruct((1, _HID), jnp.float32),
        ],
    )(parts, h, w1, w2, b1, scale, shift2)


def _readout_body(hs_ref, w0_ref, b0_ref, s0_ref, t0_ref,
                  w1_ref, b1_ref, s1_ref, t1_ref, wf_ref, bf_ref, o_ref):
    g = hs_ref[...] * (1.0 / _N)
    y = _softplus(jnp.dot(g, w0_ref[...], preferred_element_type=jnp.float32)
                  + b0_ref[...])
    y = y * s0_ref[...] + t0_ref[...]
    z = _softplus(jnp.dot(y, w1_ref[...], preferred_element_type=jnp.float32)
                  + b1_ref[...])
    z = z * s1_ref[...] + t1_ref[...]
    o_ref[...] = jnp.dot(z, wf_ref[...],
                         preferred_element_type=jnp.float32) + bf_ref[...]


def _readout(hsum, w0, b0, s0, t0, w1, b1, s1, t1, wf_pad, bf_pad):
    h2 = _HID // 2
    return pl.pallas_call(
        _readout_body,
        grid=(1,),
        in_specs=[
            pl.BlockSpec((1, _HID), lambda i: (0, 0)),
            pl.BlockSpec((_HID, h2), lambda i: (0, 0)),
            pl.BlockSpec((1, h2), lambda i: (0, 0)),
            pl.BlockSpec((1, h2), lambda i: (0, 0)),
            pl.BlockSpec((1, h2), lambda i: (0, 0)),
            pl.BlockSpec((h2, h2), lambda i: (0, 0)),
            pl.BlockSpec((1, h2), lambda i: (0, 0)),
            pl.BlockSpec((1, h2), lambda i: (0, 0)),
            pl.BlockSpec((1, h2), lambda i: (0, 0)),
            pl.BlockSpec((h2, _HID), lambda i: (0, 0)),
            pl.BlockSpec((1, _HID), lambda i: (0, 0)),
        ],
        out_specs=pl.BlockSpec((1, _HID), lambda i: (0, 0)),
        out_shape=jax.ShapeDtypeStruct((1, _HID), jnp.float32),
    )(hsum, w0, b0, s0, t0, w1, b1, s1, t1, wf_pad, bf_pad)


def kernel(x, edge_index, distances, edge_attr, params):
    f32 = jnp.float32
    npad = _EP - _E

    d_pad = jnp.concatenate([distances, jnp.full((npad,), 10.0, f32)])
    col3 = jnp.concatenate(
        [edge_index[1], jnp.zeros((npad,), edge_index.dtype)]
    ).astype(jnp.int32).reshape(_NW, _CHUNKS, _CH)
    row3 = jnp.concatenate(
        [edge_index[0], jnp.zeros((npad,), edge_index.dtype)]
    ).astype(jnp.int32).reshape(_NW, _CHUNKS, _CH)

    a = jnp.stack([p["fW1"][0] for p in params["inter"]])
    b = jnp.stack([p["fb1"] for p in params["inter"]])
    csum = jnp.stack([p["fW2"].sum(axis=1) for p in params["inter"]])
    b2 = jnp.broadcast_to(
        jnp.stack([p["fb2"].sum() for p in params["inter"]])[:, None], (3, 128))

    w_all = _wcalc(d_pad.reshape(_EP // _HID, _HID), a, b, csum, b2)
    w_all = w_all.reshape(3, _NW, _CHUNKS, _CH)

    h = _embed(x, params["emb_W"], params["emb_b"][None, :])

    hsum = None
    for li, p in enumerate(params["inter"]):
        parts = _sc_scatter(h, col3, row3, w_all[li])
        bscale = p["bn_g"] / jnp.sqrt(p["bn_v"] + _EPS)
        bshift = p["bn_b"] - p["bn_m"] * bscale
        shift2 = p["ib2"] * bscale + bshift
        h, hsum = _mlp(parts, h, p["iW1"], p["iW2"], p["ib1"][None, :],
                       bscale[None, :], shift2[None, :])

    ro = []
    for p in params["out"]:
        bscale = p["bn_g"] / jnp.sqrt(p["bn_v"] + _EPS)
        bshift = p["bn_b"] - p["bn_m"] * bscale
        ro += [p["W"], p["b"][None, :], bscale[None, :], bshift[None, :]]
    wf_pad = jnp.zeros((_HID // 2, _HID), f32).at[:, :3].set(params["final_W"])
    bf_pad = jnp.zeros((1, _HID), f32).at[0, :3].set(params["final_b"])

    out = _readout(hsum, *ro, wf_pad, bf_pad)
    return out[0, :3]
